# baseline (speedup 1.0000x reference)
import functools
import jax
import jax.numpy as jnp
from jax import lax
from jax.experimental import pallas as pl
from jax.experimental.pallas import tpu as pltpu
from jax.experimental.pallas import tpu_sc as plsc

_CHUNK = 16
_NBUF = 4


def _sc_kernel(x, pos_table):
    B, S, D = x.shape
    NC, NS = 2, 16
    NW = NC * NS
    rows_per_w = S // NW
    n_chunks = rows_per_w // _CHUNK
    n_iters = n_chunks * B
    cd = _CHUNK * D
    n_vec = cd // 16
    x1 = x.reshape(B * S * D)
    t1 = pos_table.reshape(pos_table.shape[0] * D)
    mesh = plsc.VectorSubcoreMesh(
        core_axis_name="c", subcore_axis_name="s", num_cores=NC
    )

    @functools.partial(
        pl.kernel,
        mesh=mesh,
        out_type=jax.ShapeDtypeStruct((B * S * D,), jnp.float32),
        scratch_types=[
            pltpu.VMEM((_NBUF, cd), jnp.float32),
            pltpu.VMEM((cd,), jnp.float32),
            pltpu.SemaphoreType.DMA,
            pltpu.SemaphoreType.DMA,
            pltpu.SemaphoreType.DMA,
            pltpu.SemaphoreType.DMA,
            pltpu.SemaphoreType.DMA,
            pltpu.SemaphoreType.DMA,
            pltpu.SemaphoreType.DMA,
            pltpu.SemaphoreType.DMA,
        ],
    )
    def k(x_hbm, tbl_hbm, out_hbm, acc_v, tbl_v, *sems):
        sem_ld = sems[:_NBUF]
        sem_st = sems[_NBUF:]
        wid = lax.axis_index("s") * NC + lax.axis_index("c")
        base0 = wid * rows_per_w * D

        def x_off(i):
            c, b = divmod(i, B)
            return b * S * D + base0 + c * cd

        def t_off(c):
            return base0 + c * cd

        loads = [None] * _NBUF
        stores = [None] * _NBUF
        for j in range(_NBUF - 1):
            loads[j] = pltpu.async_copy(
                x_hbm.at[pl.ds(x_off(j), cd)], acc_v.at[j], sem_ld[j]
            )
        for i in range(n_iters):
            cur = i % _NBUF
            pre = (i + _NBUF - 1) % _NBUF
            if i + _NBUF - 1 < n_iters:
                if stores[pre] is not None:
                    stores[pre].wait()
                loads[pre] = pltpu.async_copy(
                    x_hbm.at[pl.ds(x_off(i + _NBUF - 1), cd)],
                    acc_v.at[pre],
                    sem_ld[pre],
                )
            c = i // B
            if i % B == 0:
                pltpu.sync_copy(tbl_hbm.at[pl.ds(t_off(c), cd)], tbl_v)
            loads[cur].wait()
            acc = acc_v.at[cur]

            @plsc.parallel_loop(0, n_vec, unroll=8)
            def body(j):
                sl = pl.ds(j * 16, 16)
                acc[sl] = acc[sl] + tbl_v[sl]

            stores[cur] = pltpu.async_copy(
                acc_v.at[cur], out_hbm.at[pl.ds(x_off(i), cd)], sem_st[cur]
            )
        for j in range(_NBUF):
            if stores[j] is not None:
                stores[j].wait()

    out1 = k(x1, t1)
    return out1.reshape(B, S, D)


def kernel(x, pos_table):
    return _sc_kernel(x, pos_table)


# speedup vs baseline: 5.3884x; 5.3884x over previous
import jax
import jax.numpy as jnp
from jax.experimental import pallas as pl


def _add_kernel(x_ref, pos_ref, o_ref):
    o_ref[...] = x_ref[...] + pos_ref[...]


def kernel(x, pos_table):
    B, S, D = x.shape
    CS = 2048
    grid = (S // CS, B)
    return pl.pallas_call(
        _add_kernel,
        grid=grid,
        in_specs=[
            pl.BlockSpec((1, CS, D), lambda s, b: (b, s, 0)),
            pl.BlockSpec((CS, D), lambda s, b: (s, 0)),
        ],
        out_specs=pl.BlockSpec((1, CS, D), lambda s, b: (b, s, 0)),
        out_shape=jax.ShapeDtypeStruct((B, S, D), x.dtype),
    )(x, pos_table)
